# baseline (speedup 1.0000x reference)
import functools

import jax
import jax.numpy as jnp
from jax import lax
from jax.experimental import pallas as pl
from jax.experimental.pallas import tpu as pltpu
from jax.experimental.pallas import tpu_sc as plsc

EMB_DIM = 64
N_IDS = 4096 * 200
NUM_WORKERS = 32
ROWS_PER_W = N_IDS // NUM_WORKERS
CHUNK = 400
N_CHUNKS = ROWS_PER_W // CHUNK
NBUF = 4
LAG = 2
N_GROUPS = N_CHUNKS // NBUF

_mesh = plsc.VectorSubcoreMesh(core_axis_name="c", subcore_axis_name="s")

K0_ROWS = 31248
K0_RCH = 336
K0_PCH = K0_RCH // 2
K0_NCH = K0_ROWS // K0_RCH
K0_TAIL = 1000000 - 32 * K0_ROWS


@functools.partial(
    pl.kernel,
    mesh=_mesh,
    compiler_params=pltpu.CompilerParams(use_tc_tiling_on_sc=True),
    out_type=jax.ShapeDtypeStruct((500000, 128), jnp.float32),
    scratch_types=[
        pltpu.VMEM((2, K0_RCH, EMB_DIM), jnp.float32),
        pltpu.VMEM((2, K0_PCH, 2 * EMB_DIM), jnp.float32),
        pltpu.SemaphoreType.DMA,
        pltpu.SemaphoreType.DMA,
        pltpu.SemaphoreType.DMA,
        pltpu.SemaphoreType.DMA,
    ],
)
def _compact_kernel(w_hbm, out_hbm, inb, ob, *sems):
    isem = sems[:2]
    osem = sems[2:]
    wid = lax.axis_index("s") * 2 + lax.axis_index("c")
    rbase = wid * K0_ROWS
    pbase = wid * (K0_ROWS // 2)

    def in_copy(c, b, rows=K0_RCH, roff=None):
        src = rbase + c * K0_RCH if roff is None else roff
        return pltpu.make_async_copy(
            w_hbm.at[pl.ds(pl.multiple_of(src, 8), rows), :],
            inb.at[b] if rows == K0_RCH else inb.at[b].at[pl.ds(0, rows)],
            isem[b])

    def out_copy(c, b, pairs=K0_PCH, poff=None):
        dst = pbase + c * K0_PCH if poff is None else poff
        return pltpu.make_async_copy(
            ob.at[b] if pairs == K0_PCH else ob.at[b].at[pl.ds(0, pairs)],
            out_hbm.at[pl.ds(pl.multiple_of(dst, 8), pairs), :],
            osem[b])

    def compact(b, pairs=K0_PCH):
        def body(p, carry):
            for q in range(4):
                ob[b, p, pl.ds(q * 16, 16)] = inb[b, 2 * p, pl.ds(q * 16, 16)]
                ob[b, p, pl.ds(64 + q * 16, 16)] = (
                    inb[b, 2 * p + 1, pl.ds(q * 16, 16)])
            return carry
        lax.fori_loop(0, pairs, body, 0)

    in_copy(0, 0).start()

    def visit(v, j):
        @pl.when(v >= 2)
        def _():
            out_copy(v - 2, j).wait()

        @pl.when(v + 1 < K0_NCH)
        def _():
            in_copy(v + 1, 1 - j).start()

        in_copy(v, j).wait()
        compact(j)
        out_copy(v, j).start()

    def sgroup(sg, carry):
        visit(2 * sg, 0)
        visit(2 * sg + 1, 1)
        return carry

    lax.fori_loop(0, (K0_NCH - 1) // 2, sgroup, 0)

    out_copy(K0_NCH - 3, 0).wait()
    in_copy(K0_NCH - 1, 0).wait()
    compact(0)
    out_copy(K0_NCH - 1, 0).start()
    out_copy(K0_NCH - 2, 1).wait()
    out_copy(K0_NCH - 1, 0).wait()

    @pl.when(wid == 31)
    def _tail():
        in_copy(0, 1, rows=K0_TAIL, roff=32 * K0_ROWS).start()
        in_copy(0, 1, rows=K0_TAIL, roff=32 * K0_ROWS).wait()
        compact(1, pairs=K0_TAIL // 2)
        out_copy(0, 1, pairs=K0_TAIL // 2, poff=16 * K0_ROWS * 2 // 2).start()
        out_copy(0, 1, pairs=K0_TAIL // 2, poff=16 * K0_ROWS * 2 // 2).wait()


@functools.partial(
    pl.kernel,
    mesh=_mesh,
    compiler_params=pltpu.CompilerParams(use_tc_tiling_on_sc=False),
    out_type=jax.ShapeDtypeStruct((N_IDS, 2 * EMB_DIM), jnp.float32),
    scratch_types=[
        pltpu.VMEM((ROWS_PER_W,), jnp.int32),
        pltpu.VMEM((NBUF, CHUNK, EMB_DIM), jnp.float32),
        pltpu.SemaphoreType.DMA,
        pltpu.SemaphoreType.DMA,
        pltpu.SemaphoreType.DMA,
        pltpu.SemaphoreType.DMA,
        pltpu.SemaphoreType.DMA,
        pltpu.SemaphoreType.DMA,
        pltpu.SemaphoreType.DMA,
        pltpu.SemaphoreType.DMA,
    ],
)
def _gather_kernel(ids_hbm, table_hbm, out_hbm, idx_all, rows, *sems):
    gsem = sems[:NBUF]
    ssem = sems[NBUF:]
    wid = lax.axis_index("s") * 2 + lax.axis_index("c")
    base = wid * ROWS_PER_W

    def gather_copy(c, b):
        return pltpu.make_async_copy(
            table_hbm.at[idx_all.at[pl.ds(pl.multiple_of(c * CHUNK, CHUNK), CHUNK)]],
            rows.at[b],
            gsem[b],
        )

    def scatter_copy(c, b):
        return pltpu.make_async_copy(
            rows.at[b],
            out_hbm.at[pl.ds(pl.multiple_of(base + c * CHUNK, CHUNK), CHUNK),
                       pl.ds(0, EMB_DIM)],
            ssem[b],
        )

    pltpu.sync_copy(ids_hbm.at[pl.ds(pl.multiple_of(base, ROWS_PER_W), ROWS_PER_W)],
                    idx_all)

    for b in range(LAG):
        gather_copy(b, b).start()

    def group(g, carry):
        for b in range(NBUF):
            v = g * NBUF + b
            bp = (b - LAG) % NBUF

            @pl.when(g > 0)
            def _wait_buf():
                scatter_copy(v - NBUF, b).wait()

            def _issue_gather():
                gather_copy(v, b).start()

            def _retire():
                p = v - LAG
                gather_copy(p, bp).wait()
                scatter_copy(p, bp).start()

            if b >= LAG:
                _issue_gather()
                _retire()
            else:
                pl.when(g > 0)(_issue_gather)
                pl.when(g > 0)(_retire)
        return carry

    lax.fori_loop(0, N_GROUPS, group, 0)

    for k in range(LAG):
        p = N_CHUNKS - LAG + k
        bp = p % NBUF
        gather_copy(p, bp).wait()
        scatter_copy(p, bp).start()
    for b in range(NBUF):
        p = N_CHUNKS - NBUF + b
        scatter_copy(p, b).wait()


def kernel(input_ids, weight):
    ids_flat = input_ids.reshape(-1).astype(jnp.int32)
    dense = _compact_kernel(weight)
    table = dense.reshape(1000000, EMB_DIM)
    out = _gather_kernel(ids_flat, table)
    return out.reshape(4096, 200, 2 * EMB_DIM)[:, :, :EMB_DIM]


# speedup vs baseline: 1.0142x; 1.0142x over previous
import functools

import jax
import jax.numpy as jnp
from jax import lax
from jax.experimental import pallas as pl
from jax.experimental.pallas import tpu as pltpu
from jax.experimental.pallas import tpu_sc as plsc

EMB_DIM = 64
N_IDS = 4096 * 200
NUM_WORKERS = 32
ROWS_PER_W = N_IDS // NUM_WORKERS
CHUNK = 400
N_CHUNKS = ROWS_PER_W // CHUNK
NBUF = 4
LAG = 2
N_GROUPS = N_CHUNKS // NBUF

_mesh = plsc.VectorSubcoreMesh(core_axis_name="c", subcore_axis_name="s")

K0_ROWS = 31248
K0_RCH = 336
K0_PCH = K0_RCH // 2
K0_NCH = K0_ROWS // K0_RCH
K0_TAIL = 1000000 - 32 * K0_ROWS


@functools.partial(
    pl.kernel,
    mesh=_mesh,
    compiler_params=pltpu.CompilerParams(use_tc_tiling_on_sc=True),
    out_type=jax.ShapeDtypeStruct((500000, 128), jnp.float32),
    scratch_types=[
        pltpu.VMEM((2, K0_RCH, EMB_DIM), jnp.float32),
        pltpu.VMEM((2, K0_PCH, 2 * EMB_DIM), jnp.float32),
        pltpu.SemaphoreType.DMA,
        pltpu.SemaphoreType.DMA,
        pltpu.SemaphoreType.DMA,
        pltpu.SemaphoreType.DMA,
    ],
)
def _compact_kernel(w_hbm, out_hbm, inb, ob, *sems):
    isem = sems[:2]
    osem = sems[2:]
    wid = lax.axis_index("s") * 2 + lax.axis_index("c")
    rbase = wid * K0_ROWS
    pbase = wid * (K0_ROWS // 2)

    def in_copy(c, b, rows=K0_RCH, roff=None):
        src = rbase + c * K0_RCH if roff is None else roff
        return pltpu.make_async_copy(
            w_hbm.at[pl.ds(pl.multiple_of(src, 8), rows), :],
            inb.at[b] if rows == K0_RCH else inb.at[b].at[pl.ds(0, rows)],
            isem[b])

    def out_copy(c, b, pairs=K0_PCH, poff=None):
        dst = pbase + c * K0_PCH if poff is None else poff
        return pltpu.make_async_copy(
            ob.at[b] if pairs == K0_PCH else ob.at[b].at[pl.ds(0, pairs)],
            out_hbm.at[pl.ds(pl.multiple_of(dst, 8), pairs), :],
            osem[b])

    def compact(b, pairs=K0_PCH):
        def body(i, carry):
            for u in range(8):
                p = i * 8 + u
                for q in range(4):
                    ob[b, p, pl.ds(q * 16, 16)] = (
                        inb[b, 2 * p, pl.ds(q * 16, 16)])
                    ob[b, p, pl.ds(64 + q * 16, 16)] = (
                        inb[b, 2 * p + 1, pl.ds(q * 16, 16)])
            return carry
        lax.fori_loop(0, pairs // 8, body, 0)

    in_copy(0, 0).start()

    def visit(v, j):
        @pl.when(v >= 2)
        def _():
            out_copy(v - 2, j).wait()

        @pl.when(v + 1 < K0_NCH)
        def _():
            in_copy(v + 1, 1 - j).start()

        in_copy(v, j).wait()
        compact(j)
        out_copy(v, j).start()

    def sgroup(sg, carry):
        visit(2 * sg, 0)
        visit(2 * sg + 1, 1)
        return carry

    lax.fori_loop(0, (K0_NCH - 1) // 2, sgroup, 0)

    out_copy(K0_NCH - 3, 0).wait()
    in_copy(K0_NCH - 1, 0).wait()
    compact(0)
    out_copy(K0_NCH - 1, 0).start()
    out_copy(K0_NCH - 2, 1).wait()
    out_copy(K0_NCH - 1, 0).wait()

    @pl.when(wid == 31)
    def _tail():
        in_copy(0, 1, rows=K0_TAIL, roff=32 * K0_ROWS).start()
        in_copy(0, 1, rows=K0_TAIL, roff=32 * K0_ROWS).wait()
        compact(1, pairs=K0_TAIL // 2)
        out_copy(0, 1, pairs=K0_TAIL // 2, poff=16 * K0_ROWS * 2 // 2).start()
        out_copy(0, 1, pairs=K0_TAIL // 2, poff=16 * K0_ROWS * 2 // 2).wait()


@functools.partial(
    pl.kernel,
    mesh=_mesh,
    compiler_params=pltpu.CompilerParams(use_tc_tiling_on_sc=False),
    out_type=jax.ShapeDtypeStruct((N_IDS, 2 * EMB_DIM), jnp.float32),
    scratch_types=[
        pltpu.VMEM((ROWS_PER_W,), jnp.int32),
        pltpu.VMEM((NBUF, CHUNK, EMB_DIM), jnp.float32),
        pltpu.SemaphoreType.DMA,
        pltpu.SemaphoreType.DMA,
        pltpu.SemaphoreType.DMA,
        pltpu.SemaphoreType.DMA,
        pltpu.SemaphoreType.DMA,
        pltpu.SemaphoreType.DMA,
        pltpu.SemaphoreType.DMA,
        pltpu.SemaphoreType.DMA,
    ],
)
def _gather_kernel(ids_hbm, table_hbm, out_hbm, idx_all, rows, *sems):
    gsem = sems[:NBUF]
    ssem = sems[NBUF:]
    wid = lax.axis_index("s") * 2 + lax.axis_index("c")
    base = wid * ROWS_PER_W

    def gather_copy(c, b):
        return pltpu.make_async_copy(
            table_hbm.at[idx_all.at[pl.ds(pl.multiple_of(c * CHUNK, CHUNK), CHUNK)]],
            rows.at[b],
            gsem[b],
        )

    def scatter_copy(c, b):
        return pltpu.make_async_copy(
            rows.at[b],
            out_hbm.at[pl.ds(pl.multiple_of(base + c * CHUNK, CHUNK), CHUNK),
                       pl.ds(0, EMB_DIM)],
            ssem[b],
        )

    pltpu.sync_copy(ids_hbm.at[pl.ds(pl.multiple_of(base, ROWS_PER_W), ROWS_PER_W)],
                    idx_all)

    for b in range(LAG):
        gather_copy(b, b).start()

    def group(g, carry):
        for b in range(NBUF):
            v = g * NBUF + b
            bp = (b - LAG) % NBUF

            @pl.when(g > 0)
            def _wait_buf():
                scatter_copy(v - NBUF, b).wait()

            def _issue_gather():
                gather_copy(v, b).start()

            def _retire():
                p = v - LAG
                gather_copy(p, bp).wait()
                scatter_copy(p, bp).start()

            if b >= LAG:
                _issue_gather()
                _retire()
            else:
                pl.when(g > 0)(_issue_gather)
                pl.when(g > 0)(_retire)
        return carry

    lax.fori_loop(0, N_GROUPS, group, 0)

    for k in range(LAG):
        p = N_CHUNKS - LAG + k
        bp = p % NBUF
        gather_copy(p, bp).wait()
        scatter_copy(p, bp).start()
    for b in range(NBUF):
        p = N_CHUNKS - NBUF + b
        scatter_copy(p, b).wait()


def kernel(input_ids, weight):
    ids_flat = input_ids.reshape(-1).astype(jnp.int32)
    dense = _compact_kernel(weight)
    table = dense.reshape(1000000, EMB_DIM)
    out = _gather_kernel(ids_flat, table)
    return out.reshape(4096, 200, 2 * EMB_DIM)[:, :, :EMB_DIM]


# speedup vs baseline: 1.2016x; 1.1848x over previous
import functools

import jax
import jax.numpy as jnp
from jax import lax
from jax.experimental import pallas as pl
from jax.experimental.pallas import tpu as pltpu
from jax.experimental.pallas import tpu_sc as plsc

EMB_DIM = 64
N_IDS = 4096 * 200
NUM_WORKERS = 32
ROWS_PER_W = N_IDS // NUM_WORKERS
CHUNK = 400
N_CHUNKS = ROWS_PER_W // CHUNK
NBUF = 4
LAG = 2
N_GROUPS = N_CHUNKS // NBUF

_mesh = plsc.VectorSubcoreMesh(core_axis_name="c", subcore_axis_name="s")


@functools.partial(
    pl.kernel,
    mesh=_mesh,
    compiler_params=pltpu.CompilerParams(use_tc_tiling_on_sc=False),
    out_type=jax.ShapeDtypeStruct((N_IDS, 2 * EMB_DIM), jnp.float32),
    scratch_types=[
        pltpu.VMEM((ROWS_PER_W,), jnp.int32),
        pltpu.VMEM((NBUF, CHUNK, EMB_DIM), jnp.float32),
        pltpu.SemaphoreType.DMA,
        pltpu.SemaphoreType.DMA,
        pltpu.SemaphoreType.DMA,
        pltpu.SemaphoreType.DMA,
        pltpu.SemaphoreType.DMA,
        pltpu.SemaphoreType.DMA,
        pltpu.SemaphoreType.DMA,
        pltpu.SemaphoreType.DMA,
    ],
)
def _gather_kernel(ids_hbm, table_hbm, out_hbm, idx_all, rows, *sems):
    gsem = sems[:NBUF]
    ssem = sems[NBUF:]
    wid = lax.axis_index("s") * 2 + lax.axis_index("c")
    base = wid * ROWS_PER_W

    def gather_copy(c, b):
        return pltpu.make_async_copy(
            table_hbm.at[idx_all.at[pl.ds(pl.multiple_of(c * CHUNK, CHUNK), CHUNK)]],
            rows.at[b],
            gsem[b],
        )

    def scatter_copy(c, b):
        return pltpu.make_async_copy(
            rows.at[b],
            out_hbm.at[pl.ds(pl.multiple_of(base + c * CHUNK, CHUNK), CHUNK),
                       pl.ds(0, EMB_DIM)],
            ssem[b],
        )

    pltpu.sync_copy(ids_hbm.at[pl.ds(pl.multiple_of(base, ROWS_PER_W), ROWS_PER_W)],
                    idx_all)

    for b in range(LAG):
        gather_copy(b, b).start()

    def group(g, carry):
        for b in range(NBUF):
            v = g * NBUF + b
            bp = (b - LAG) % NBUF

            @pl.when(g > 0)
            def _wait_buf():
                scatter_copy(v - NBUF, b).wait()

            def _issue_gather():
                gather_copy(v, b).start()

            def _retire():
                p = v - LAG
                gather_copy(p, bp).wait()
                scatter_copy(p, bp).start()

            if b >= LAG:
                _issue_gather()
                _retire()
            else:
                pl.when(g > 0)(_issue_gather)
                pl.when(g > 0)(_retire)
        return carry

    lax.fori_loop(0, N_GROUPS, group, 0)

    for k in range(LAG):
        p = N_CHUNKS - LAG + k
        bp = p % NBUF
        gather_copy(p, bp).wait()
        scatter_copy(p, bp).start()
    for b in range(NBUF):
        p = N_CHUNKS - NBUF + b
        scatter_copy(p, b).wait()


def kernel(input_ids, weight):
    ids_flat = input_ids.reshape(-1).astype(jnp.int32)
    out = _gather_kernel(ids_flat, weight)
    return out.reshape(4096, 200, 2 * EMB_DIM)[:, :, :EMB_DIM]


# speedup vs baseline: 1.2046x; 1.0026x over previous
import functools

import jax
import jax.numpy as jnp
from jax import lax
from jax.experimental import pallas as pl
from jax.experimental.pallas import tpu as pltpu
from jax.experimental.pallas import tpu_sc as plsc

EMB_DIM = 64
N_IDS = 4096 * 200
NUM_WORKERS = 32
ROWS_PER_W = N_IDS // NUM_WORKERS
CHUNK = 320
N_CHUNKS = ROWS_PER_W // CHUNK
NBUF = 5
LAG = 2
N_GROUPS = N_CHUNKS // NBUF

_mesh = plsc.VectorSubcoreMesh(core_axis_name="c", subcore_axis_name="s")


@functools.partial(
    pl.kernel,
    mesh=_mesh,
    compiler_params=pltpu.CompilerParams(use_tc_tiling_on_sc=False),
    out_type=jax.ShapeDtypeStruct((N_IDS, 2 * EMB_DIM), jnp.float32),
    scratch_types=[
        pltpu.VMEM((ROWS_PER_W,), jnp.int32),
        pltpu.VMEM((NBUF, CHUNK, EMB_DIM), jnp.float32),
    ] + [pltpu.SemaphoreType.DMA] * (2 * NBUF),
)
def _gather_kernel(ids_hbm, table_hbm, out_hbm, idx_all, rows, *sems):
    gsem = sems[:NBUF]
    ssem = sems[NBUF:]
    wid = lax.axis_index("s") * 2 + lax.axis_index("c")
    base = wid * ROWS_PER_W

    def gather_copy(c, b):
        return pltpu.make_async_copy(
            table_hbm.at[idx_all.at[pl.ds(pl.multiple_of(c * CHUNK, CHUNK), CHUNK)]],
            rows.at[b],
            gsem[b],
        )

    def scatter_copy(c, b):
        return pltpu.make_async_copy(
            rows.at[b],
            out_hbm.at[pl.ds(pl.multiple_of(base + c * CHUNK, CHUNK), CHUNK),
                       pl.ds(0, EMB_DIM)],
            ssem[b],
        )

    pltpu.sync_copy(ids_hbm.at[pl.ds(pl.multiple_of(base, ROWS_PER_W), ROWS_PER_W)],
                    idx_all)

    for b in range(LAG):
        gather_copy(b, b).start()

    def group(g, carry):
        for b in range(NBUF):
            v = g * NBUF + b
            bp = (b - LAG) % NBUF

            @pl.when(g > 0)
            def _wait_buf():
                scatter_copy(v - NBUF, b).wait()

            def _issue_gather():
                gather_copy(v, b).start()

            def _retire():
                p = v - LAG
                gather_copy(p, bp).wait()
                scatter_copy(p, bp).start()

            if b >= LAG:
                _issue_gather()
                _retire()
            else:
                pl.when(g > 0)(_issue_gather)
                pl.when(g > 0)(_retire)
        return carry

    lax.fori_loop(0, N_GROUPS, group, 0)

    for k in range(LAG):
        p = N_CHUNKS - LAG + k
        bp = p % NBUF
        gather_copy(p, bp).wait()
        scatter_copy(p, bp).start()
    for b in range(NBUF):
        p = N_CHUNKS - NBUF + b
        scatter_copy(p, b).wait()


def kernel(input_ids, weight):
    ids_flat = input_ids.reshape(-1).astype(jnp.int32)
    out = _gather_kernel(ids_flat, weight)
    return out.reshape(4096, 200, 2 * EMB_DIM)[:, :, :EMB_DIM]
